# baseline (speedup 1.0000x reference)
import functools

import jax
import jax.numpy as jnp
from jax import lax
from jax.experimental import pallas as pl
from jax.experimental.pallas import tpu as pltpu
from jax.experimental.pallas import tpu_sc as plsc

BATCH = 16384
TASKS = 16
NC = 2
NS = 16
NW = NC * NS
BPW = BATCH // NW
CHUNK = 128
NCHUNK = BPW // CHUNK
SPB = 16
BLKW = 128
DATA = 1000000
PPW = BPW // 2


def _softplus16(w):
    u = jnp.exp(-jnp.abs(w))
    z = u / (u + 2.0)
    z2 = z * z
    poly = 1.0 + z2 * (1.0 / 3.0 + z2 * (1.0 / 5.0 + z2 * (1.0 / 7.0 + z2 * (1.0 / 9.0))))
    return jnp.maximum(w, 0.0) + 2.0 * z * poly


def _sc_body(llo_hbm, lhi_hbm, idx_hbm, col_hbm, table_hbm, out_hbm,
             idx_v, col_v, llo_v, lhi_v, rows_a, rows_b, acc_v,
             gsem_a, gsem_b, lsem):
    wid = lax.axis_index("s") * NC + lax.axis_index("c")
    pltpu.sync_copy(idx_hbm.at[wid], idx_v)
    l0cp = pltpu.async_copy(llo_hbm.at[wid], llo_v, lsem)
    l1cp = pltpu.async_copy(lhi_hbm.at[wid], lhi_v, lsem)
    ccp = pltpu.async_copy(col_hbm.at[wid], col_v, lsem)
    bufs = [rows_a, rows_b]
    sems = [gsem_a, gsem_b]
    cps = [None] * NCHUNK
    cps[0] = pltpu.async_copy(table_hbm.at[idx_v.at[0]], rows_a, gsem_a)
    ccp.wait()
    l0cp.wait()
    l1cp.wait()

    lane = lax.iota(jnp.int32, 16)
    hi_half = (lane >= 8).astype(jnp.int32)
    himask = jnp.full((16,), -65536, jnp.int32)
    acc = jnp.zeros((TASKS,), jnp.float32)
    for j in range(NCHUNK):
        if j + 1 < NCHUNK:
            cps[j + 1] = pltpu.async_copy(
                table_hbm.at[idx_v.at[j + 1]], bufs[(j + 1) % 2],
                sems[(j + 1) % 2])
        cps[j].wait()
        buf = bufs[j % 2]
        pbase = j * (CHUNK // 2)

        def body(k, acc, buf=buf, pbase=pbase):
            terms = []
            for t in range(2):
                p = k * 2 + t
                off = pl.multiple_of((pbase + p) * 16, 16)
                cols = col_v[pl.ds(off, 16)]
                rows = hi_half + 2 * p
                x = plsc.load_gather(buf, [rows, cols])
                wlo = lax.bitcast_convert_type(
                    lax.shift_left(x, 16), jnp.float32)
                whi = lax.bitcast_convert_type(
                    lax.bitwise_and(x, himask), jnp.float32)
                llo = llo_v[pl.ds(off, 16)]
                lhi = lhi_v[pl.ds(off, 16)]
                terms.append(_softplus16(wlo) * llo + _softplus16(whi) * lhi)
            return acc + (terms[0] + terms[1])

        acc = lax.fori_loop(0, CHUNK // 4, body, acc)
    acc_v[...] = acc * (1.0 / BATCH)
    pltpu.sync_copy(acc_v, out_hbm.at[wid])


_sc_partials = functools.partial(
    pl.kernel,
    out_type=jax.ShapeDtypeStruct((NW, TASKS), jnp.float32),
    mesh=plsc.VectorSubcoreMesh(core_axis_name="c", subcore_axis_name="s"),
    compiler_params=pltpu.CompilerParams(needs_layout_passes=False),
    scratch_types=[
        pltpu.VMEM((NCHUNK, CHUNK), jnp.int32),
        pltpu.VMEM((PPW * 16,), jnp.int32),
        pltpu.VMEM((PPW * 16,), jnp.float32),
        pltpu.VMEM((PPW * 16,), jnp.float32),
        pltpu.VMEM((CHUNK, BLKW), jnp.int32),
        pltpu.VMEM((CHUNK, BLKW), jnp.int32),
        pltpu.VMEM((TASKS,), jnp.float32),
        pltpu.SemaphoreType.DMA,
        pltpu.SemaphoreType.DMA,
        pltpu.SemaphoreType.DMA,
    ],
)(_sc_body)


def _tc_sum_body(x_ref, o_ref):
    o_ref[0, 0] = jnp.sum(x_ref[...])


_tc_sum = pl.pallas_call(
    _tc_sum_body,
    out_shape=jax.ShapeDtypeStruct((1, 1), jnp.float32),
    out_specs=pl.BlockSpec(memory_space=pltpu.SMEM),
)


def kernel(losses, sample_id, weights_table):
    sid = sample_id.astype(jnp.int32)
    idx = jnp.reshape(sid // SPB, (NW, NCHUNK, CHUNK))
    base = (sid % SPB) * 8
    cols = base.reshape(BATCH // 2, 2, 1) + jnp.arange(8, dtype=jnp.int32)
    cols = jnp.reshape(cols, (NW, PPW * 16))
    l4 = losses.reshape(BATCH // 2, 2, 8, 2)
    llo = jnp.reshape(l4[:, :, :, 0], (NW, PPW * 16))
    lhi = jnp.reshape(l4[:, :, :, 1], (NW, PPW * 16))
    wt16 = weights_table.astype(jnp.bfloat16)
    wpacked = jax.lax.bitcast_convert_type(
        wt16.reshape(DATA, 8, 2), jnp.int32)
    table_r = jnp.reshape(wpacked, (DATA // SPB, BLKW))
    partials = _sc_partials(llo, lhi, idx, cols, table_r)
    total = _tc_sum(partials)
    return total[0, 0]


# speedup vs baseline: 2.6554x; 2.6554x over previous
import functools

import jax
import jax.numpy as jnp
from jax import lax
from jax.experimental import pallas as pl
from jax.experimental.pallas import tpu as pltpu
from jax.experimental.pallas import tpu_sc as plsc

BATCH = 16384
TASKS = 16
NC = 2
NS = 16
NW = NC * NS
BPW = BATCH // NW
CHUNK = 128
NCHUNK = BPW // CHUNK
ROWS_PER_BLK = 8
BLKW = ROWS_PER_BLK * TASKS
DATA = 1000000


def _softplus16(w):
    u = jnp.exp(-jnp.abs(w))
    z = u / (u + 2.0)
    z2 = z * z
    poly = 1.0 + z2 * (1.0 / 3.0 + z2 * (1.0 / 5.0 + z2 * (1.0 / 7.0 + z2 * (1.0 / 9.0))))
    return jnp.maximum(w, 0.0) + 2.0 * z * poly


def _sc_body(loss_hbm, idx_hbm, col_hbm, table_hbm, out_hbm,
             idx_v, col_v, loss_v, rows_a, rows_b, acc_v,
             gsem_a, gsem_b, lsem):
    wid = lax.axis_index("s") * NC + lax.axis_index("c")
    pltpu.sync_copy(idx_hbm.at[wid], idx_v)
    lcp = pltpu.async_copy(loss_hbm.at[wid], loss_v, lsem)
    ccp = pltpu.async_copy(col_hbm.at[wid], col_v, lsem)
    bufs = [rows_a, rows_b]
    sems = [gsem_a, gsem_b]
    cps = [None] * NCHUNK
    cps[0] = pltpu.async_copy(table_hbm.at[idx_v.at[0]], rows_a, gsem_a)
    ccp.wait()
    lcp.wait()

    acc = jnp.zeros((TASKS,), jnp.float32)
    for j in range(NCHUNK):
        if j + 1 < NCHUNK:
            cps[j + 1] = pltpu.async_copy(
                table_hbm.at[idx_v.at[j + 1]], bufs[(j + 1) % 2],
                sems[(j + 1) % 2])
        cps[j].wait()
        buf = bufs[j % 2]
        base = j * CHUNK

        def body(k, acc, buf=buf, base=base):
            r = k * 4
            terms = []
            for t in range(4):
                off = pl.multiple_of((base + r + t) * TASKS, 16)
                cols = col_v[pl.ds(off, TASKS)]
                rows = jnp.full((TASKS,), r + t, jnp.int32)
                w = plsc.load_gather(buf, [rows, cols])
                l = loss_v[pl.ds(off, TASKS)]
                terms.append(_softplus16(w) * l)
            return acc + ((terms[0] + terms[1]) + (terms[2] + terms[3]))

        acc = lax.fori_loop(0, CHUNK // 4, body, acc)
    acc_v[...] = acc * (1.0 / BATCH)
    pltpu.sync_copy(acc_v, out_hbm.at[wid])


_sc_partials = functools.partial(
    pl.kernel,
    out_type=jax.ShapeDtypeStruct((NW, TASKS), jnp.float32),
    mesh=plsc.VectorSubcoreMesh(core_axis_name="c", subcore_axis_name="s"),
    compiler_params=pltpu.CompilerParams(needs_layout_passes=False),
    scratch_types=[
        pltpu.VMEM((NCHUNK, CHUNK), jnp.int32),
        pltpu.VMEM((BPW * TASKS,), jnp.int32),
        pltpu.VMEM((BPW * TASKS,), jnp.float32),
        pltpu.VMEM((CHUNK, BLKW), jnp.float32),
        pltpu.VMEM((CHUNK, BLKW), jnp.float32),
        pltpu.VMEM((TASKS,), jnp.float32),
        pltpu.SemaphoreType.DMA,
        pltpu.SemaphoreType.DMA,
        pltpu.SemaphoreType.DMA,
    ],
)(_sc_body)


def _tc_sum_body(x_ref, o_ref):
    o_ref[0, 0] = jnp.sum(x_ref[...])


_tc_sum = pl.pallas_call(
    _tc_sum_body,
    out_shape=jax.ShapeDtypeStruct((1, 1), jnp.float32),
    out_specs=pl.BlockSpec(memory_space=pltpu.SMEM),
)


def kernel(losses, sample_id, weights_table):
    sid = sample_id.astype(jnp.int32)
    idx = jnp.reshape(sid // ROWS_PER_BLK, (NW, NCHUNK, CHUNK))
    cols = ((sid % ROWS_PER_BLK) * TASKS)[:, None] + jnp.arange(
        TASKS, dtype=jnp.int32)[None, :]
    cols = jnp.reshape(cols, (NW, BPW * TASKS))
    loss_r = jnp.reshape(losses, (NW, BPW * TASKS))
    table_r = jnp.reshape(weights_table, (DATA // ROWS_PER_BLK, BLKW))
    partials = _sc_partials(loss_r, idx, cols, table_r)
    total = _tc_sum(partials)
    return total[0, 0]
